# baseline (speedup 1.0000x reference)
import functools

import jax
import jax.numpy as jnp
from jax.experimental import pallas as pl
from jax.experimental.pallas import tpu as pltpu

_EPS = 1e-5
_CCHUNK = 8
_VMEM_LIMIT_BYTES = 64 * 1024 * 1024


def _rms_tile(x_ref, o_ref, *, inv_n: float, eps: float):
    x = x_ref[...]
    x2 = x * x
    s = jnp.sum(x2, axis=2, keepdims=True)
    s_down = jnp.roll(s, -1, axis=1)
    s_up = jnp.roll(s, 1, axis=1)
    h = jax.lax.broadcasted_iota(jnp.int32, s.shape, 1)
    pair = s + jnp.where(h % 2 == 0, s_down, s_up)
    o_ref[...] = x * jax.lax.rsqrt(pair * inv_n + eps)


def _core_body(x_hbm, o_hbm, *, features: int, dims, inv_n: float, eps: float):
    dim1, dim2, batch = dims
    half = features // 2
    core = pl.program_id(0)
    xs = x_hbm.at[pl.ds(core * half, half)]
    os_ = o_hbm.at[pl.ds(core * half, half)]
    tile = functools.partial(_rms_tile, inv_n=inv_n, eps=eps)
    pltpu.emit_pipeline(
        tile,
        grid=(half // _CCHUNK,),
        in_specs=[pl.BlockSpec((_CCHUNK, dim1, dim2, batch),
                               lambda i: (i, 0, 0, 0))],
        out_specs=[pl.BlockSpec((_CCHUNK, dim1, dim2, batch),
                                lambda i: (i, 0, 0, 0))],
    )(xs, os_)


def kernel(x):
    batch, features, dim1, dim2 = x.shape
    n = features
    assert n == 2 * dim2, "group = 2 consecutive rows of the last dim"
    assert (features // 2) % _CCHUNK == 0

    xt = jnp.transpose(x, (1, 2, 3, 0))

    body = functools.partial(
        _core_body, features=features, dims=(dim1, dim2, batch),
        inv_n=1.0 / float(n), eps=_EPS,
    )
    yt = pl.pallas_call(
        body,
        out_shape=jax.ShapeDtypeStruct(xt.shape, x.dtype),
        grid=(2,),
        in_specs=[pl.BlockSpec(memory_space=pltpu.MemorySpace.HBM)],
        out_specs=pl.BlockSpec(memory_space=pltpu.MemorySpace.HBM),
        compiler_params=pltpu.CompilerParams(
            dimension_semantics=("parallel",),
            vmem_limit_bytes=_VMEM_LIMIT_BYTES,
        ),
    )(xt)
    return jnp.transpose(yt, (3, 0, 1, 2))


# speedup vs baseline: 1.1663x; 1.1663x over previous
import functools

import jax
import jax.numpy as jnp
from jax.experimental import pallas as pl
from jax.experimental.pallas import tpu as pltpu

_EPS = 1e-5
_CCHUNK = 16
_VMEM_LIMIT_BYTES = 64 * 1024 * 1024


def _rms_tile(x_ref, o_ref, *, inv_n: float, eps: float):
    x = x_ref[...]
    x2 = x * x
    s = jnp.sum(x2, axis=2, keepdims=True)
    s_down = jnp.roll(s, -1, axis=1)
    s_up = jnp.roll(s, 1, axis=1)
    h = jax.lax.broadcasted_iota(jnp.int32, s.shape, 1)
    pair = s + jnp.where(h % 2 == 0, s_down, s_up)
    o_ref[...] = x * jax.lax.rsqrt(pair * inv_n + eps)


def _core_body(x_hbm, o_hbm, *, features: int, dims, inv_n: float, eps: float):
    dim1, dim2, batch = dims
    half = features // 2
    core = pl.program_id(0)
    xs = x_hbm.at[pl.ds(core * half, half)]
    os_ = o_hbm.at[pl.ds(core * half, half)]
    tile = functools.partial(_rms_tile, inv_n=inv_n, eps=eps)
    pltpu.emit_pipeline(
        tile,
        grid=(half // _CCHUNK,),
        in_specs=[pl.BlockSpec((_CCHUNK, dim1, dim2, batch),
                               lambda i: (i, 0, 0, 0))],
        out_specs=[pl.BlockSpec((_CCHUNK, dim1, dim2, batch),
                                lambda i: (i, 0, 0, 0))],
    )(xs, os_)


def kernel(x):
    batch, features, dim1, dim2 = x.shape
    n = features
    assert n == 2 * dim2, "group = 2 consecutive rows of the last dim"
    assert (features // 2) % _CCHUNK == 0

    xt = jnp.transpose(x, (1, 2, 3, 0))

    body = functools.partial(
        _core_body, features=features, dims=(dim1, dim2, batch),
        inv_n=1.0 / float(n), eps=_EPS,
    )
    yt = pl.pallas_call(
        body,
        out_shape=jax.ShapeDtypeStruct(xt.shape, x.dtype),
        grid=(2,),
        in_specs=[pl.BlockSpec(memory_space=pltpu.MemorySpace.HBM)],
        out_specs=pl.BlockSpec(memory_space=pltpu.MemorySpace.HBM),
        compiler_params=pltpu.CompilerParams(
            dimension_semantics=("parallel",),
            vmem_limit_bytes=_VMEM_LIMIT_BYTES,
        ),
    )(xt)
    return jnp.transpose(yt, (3, 0, 1, 2))
